# baseline (speedup 1.0000x reference)
import functools

import jax
import jax.numpy as jnp
import numpy as np
from jax import lax
from jax.experimental import pallas as pl
from jax.experimental.pallas import tpu as pltpu
from jax.experimental.pallas import tpu_sc as plsc

NC, NS, L = 2, 16, 16
NW = NC * NS
B, SEQ, D = 1024, 200, 128
V = 50176
BPW = B // NW
CHA, CHB = 104, 96
DW = D // 4
ND = DW // L

_mesh = plsc.VectorSubcoreMesh(
    core_axis_name="c", subcore_axis_name="s", num_cores=NC, num_subcores=NS
)


@functools.partial(
    pl.kernel,
    out_type=jax.ShapeDtypeStruct((B, D), jnp.float32),
    mesh=_mesh,
    compiler_params=pltpu.CompilerParams(use_tc_tiling_on_sc=False),
    scratch_types=[
        pltpu.VMEM((BPW, SEQ), jnp.int32),
        pltpu.VMEM((CHA, DW), jnp.int32),
        pltpu.VMEM((CHB, DW), jnp.int32),
        pltpu.VMEM((BPW, D), jnp.float32),
        pltpu.SemaphoreType.DMA,
        pltpu.SemaphoreType.DMA,
    ],
)
def _encode(x_hbm, table_hbm, out_hbm, idx_v, rowsa, rowsb, out_v, sema, semb):
    wid = lax.axis_index("s") * NC + lax.axis_index("c")

    pltpu.sync_copy(x_hbm.at[pl.ds(wid * BPW, BPW)], idx_v)

    zero8 = tuple(jnp.zeros((L,), jnp.int32) for _ in range(ND))

    def fire_a(s):
        pltpu.async_copy(table_hbm.at[idx_v.at[s, pl.ds(0, CHA)]], rowsa, sema)

    def fire_b(s):
        pltpu.async_copy(table_hbm.at[idx_v.at[s, pl.ds(CHA, CHB)]], rowsb, semb)

    def wait_a(s):
        pltpu.make_async_copy(
            table_hbm.at[idx_v.at[s, pl.ds(0, CHA)]], rowsa, sema).wait()

    def wait_b(s):
        pltpu.make_async_copy(
            table_hbm.at[idx_v.at[s, pl.ds(CHA, CHB)]], rowsb, semb).wait()

    def reduce_chunk(buf, n, acc8):
        def row_body(r, a):
            return tuple(a[h] + buf[r, pl.ds(h * L, L)] for h in range(ND))

        return lax.fori_loop(0, n, row_body, acc8)

    fire_a(0)
    fire_b(0)

    def sample_body(s, carry):
        wait_a(s)
        acc8 = reduce_chunk(rowsa, CHA, zero8)

        @pl.when(s + 1 < BPW)
        def _():
            fire_a(s + 1)

        wait_b(s)
        acc8 = reduce_chunk(rowsb, CHB, acc8)

        @pl.when(s + 1 < BPW)
        def _():
            fire_b(s + 1)

        for h in range(ND):
            for k in range(4):
                neg = (acc8[h] >> (8 * k)) & 0xFF
                sv = SEQ - 2 * neg
                out_v[s, pl.ds(32 * k + 16 * h, L)] = sv.astype(jnp.float32)
        return carry

    lax.fori_loop(0, BPW, sample_body, 0)
    pltpu.sync_copy(out_v, out_hbm.at[pl.ds(wid * BPW, BPW)])


def kernel(x, table):
    x2 = x.astype(jnp.int32)
    sgn = (table < 0).astype(jnp.int32)
    q = [sgn[:, 32 * k : 32 * k + 32] for k in range(4)]
    tw = q[0] | (q[1] << 8) | (q[2] << 16) | (q[3] << 24)
    return _encode(x2, tw)


# speedup vs baseline: 1.1300x; 1.1300x over previous
import functools

import jax
import jax.numpy as jnp
import numpy as np
from jax import lax
from jax.experimental import pallas as pl
from jax.experimental.pallas import tpu as pltpu
from jax.experimental.pallas import tpu_sc as plsc

NC, NS, L = 2, 16, 16
NW = NC * NS
B, SEQ, D = 1024, 200, 128
V = 50176
BPW = B // NW
CHA, CHB = 104, 96
DW = D // 4
ND = DW // L

_mesh = plsc.VectorSubcoreMesh(
    core_axis_name="c", subcore_axis_name="s", num_cores=NC, num_subcores=NS
)


@functools.partial(
    pl.kernel,
    out_type=jax.ShapeDtypeStruct((B, D), jnp.float32),
    mesh=_mesh,
    compiler_params=pltpu.CompilerParams(use_tc_tiling_on_sc=False),
    scratch_types=[
        pltpu.VMEM((BPW, SEQ), jnp.int32),
        pltpu.VMEM((CHA, DW), jnp.int32),
        pltpu.VMEM((CHB, DW), jnp.int32),
        pltpu.VMEM((BPW, D), jnp.float32),
        pltpu.SemaphoreType.DMA,
        pltpu.SemaphoreType.DMA,
    ],
)
def _encode(x_hbm, table_hbm, out_hbm, idx_v, rowsa, rowsb, out_v, sema, semb):
    wid = lax.axis_index("s") * NC + lax.axis_index("c")

    pltpu.sync_copy(x_hbm.at[pl.ds(wid * BPW, BPW)], idx_v)

    zero8 = tuple(jnp.zeros((L,), jnp.int32) for _ in range(ND))

    def fire_a(s):
        pltpu.async_copy(table_hbm.at[idx_v.at[s, pl.ds(0, CHA)]], rowsa, sema)

    def fire_b(s):
        pltpu.async_copy(table_hbm.at[idx_v.at[s, pl.ds(CHA, CHB)]], rowsb, semb)

    def wait_a(s):
        pltpu.make_async_copy(
            table_hbm.at[idx_v.at[s, pl.ds(0, CHA)]], rowsa, sema).wait()

    def wait_b(s):
        pltpu.make_async_copy(
            table_hbm.at[idx_v.at[s, pl.ds(CHA, CHB)]], rowsb, semb).wait()

    def reduce_chunk(buf, n, acc8):
        def row_body(r, a):
            return tuple(a[h] + buf[r, pl.ds(h * L, L)] for h in range(ND))

        return lax.fori_loop(0, n, row_body, acc8)

    fire_a(0)
    fire_b(0)

    def sample_body(s, carry):
        wait_a(s)
        acc8 = reduce_chunk(rowsa, CHA, zero8)

        @pl.when(s + 1 < BPW)
        def _():
            fire_a(s + 1)

        wait_b(s)
        acc8 = reduce_chunk(rowsb, CHB, acc8)

        @pl.when(s + 1 < BPW)
        def _():
            fire_b(s + 1)

        for h in range(ND):
            for k in range(4):
                neg = (acc8[h] >> (8 * k)) & 0xFF
                sv = SEQ - 2 * neg
                out_v[s, pl.ds(32 * k + 16 * h, L)] = sv.astype(jnp.float32)
        return carry

    lax.fori_loop(0, BPW, sample_body, 0)
    pltpu.sync_copy(out_v, out_hbm.at[pl.ds(wid * BPW, BPW)])


_Q0 = np.zeros((D, 32), np.float32)
_Q1 = np.zeros((D, 32), np.float32)
for _m in range(32):
    _Q0[_m, _m] = 1.0
    _Q0[32 + _m, _m] = 256.0
    _Q1[64 + _m, _m] = 1.0
    _Q1[96 + _m, _m] = 256.0


def kernel(x, table):
    x2 = x.astype(jnp.int32)
    lo = 128.5 - 0.5 * (table @ _Q0)
    hi = 128.5 - 0.5 * (table @ _Q1)
    tw = lo.astype(jnp.int32) | (hi.astype(jnp.int32) << 16)
    return _encode(x2, tw)


# speedup vs baseline: 1.1326x; 1.0023x over previous
import functools

import jax
import jax.numpy as jnp
import numpy as np
from jax import lax
from jax.experimental import pallas as pl
from jax.experimental.pallas import tpu as pltpu
from jax.experimental.pallas import tpu_sc as plsc

NC, NS, L = 2, 16, 16
NW = NC * NS
B, SEQ, D = 1024, 200, 128
V = 50176
BPW = B // NW
CHA, CHB = 104, 96
DW = D // 4
ND = DW // L

_mesh = plsc.VectorSubcoreMesh(
    core_axis_name="c", subcore_axis_name="s", num_cores=NC, num_subcores=NS
)


@functools.partial(
    pl.kernel,
    out_type=jax.ShapeDtypeStruct((B, D), jnp.float32),
    mesh=_mesh,
    compiler_params=pltpu.CompilerParams(use_tc_tiling_on_sc=False),
    scratch_types=[
        pltpu.VMEM((BPW, SEQ), jnp.int32),
        pltpu.VMEM((CHA, DW), jnp.int32),
        pltpu.VMEM((CHB, DW), jnp.int32),
        pltpu.VMEM((BPW, D), jnp.float32),
        pltpu.SemaphoreType.DMA,
        pltpu.SemaphoreType.DMA,
    ],
)
def _encode(x_hbm, table_hbm, out_hbm, idx_v, rowsa, rowsb, out_v, sema, semb):
    wid = lax.axis_index("s") * NC + lax.axis_index("c")

    pltpu.sync_copy(x_hbm.at[pl.ds(wid * BPW, BPW)], idx_v)

    zero8 = tuple(jnp.zeros((L,), jnp.int32) for _ in range(ND))

    def fire_a(s):
        pltpu.async_copy(table_hbm.at[idx_v.at[s, pl.ds(0, CHA)]], rowsa, sema)

    def fire_b(s):
        pltpu.async_copy(table_hbm.at[idx_v.at[s, pl.ds(CHA, CHB)]], rowsb, semb)

    def wait_a(s):
        pltpu.make_async_copy(
            table_hbm.at[idx_v.at[s, pl.ds(0, CHA)]], rowsa, sema).wait()

    def wait_b(s):
        pltpu.make_async_copy(
            table_hbm.at[idx_v.at[s, pl.ds(CHA, CHB)]], rowsb, semb).wait()

    def reduce_chunk(buf, n, acc8):
        def row_body(r, a):
            return tuple(a[h] + buf[r, pl.ds(h * L, L)] for h in range(ND))

        return lax.fori_loop(0, n, row_body, acc8)

    fire_a(0)
    fire_b(0)

    def sample_body(s, carry):
        wait_a(s)
        acc8 = reduce_chunk(rowsa, CHA, zero8)

        @pl.when(s + 1 < BPW)
        def _():
            fire_a(s + 1)

        wait_b(s)
        acc8 = reduce_chunk(rowsb, CHB, acc8)

        @pl.when(s + 1 < BPW)
        def _():
            fire_b(s + 1)

        for h in range(ND):
            for k in range(4):
                neg = (acc8[h] >> (8 * k)) & 0xFF
                sv = SEQ - 2 * neg
                out_v[s, pl.ds(32 * k + 16 * h, L)] = sv.astype(jnp.float32)
        return carry

    lax.fori_loop(0, BPW, sample_body, 0)
    pltpu.sync_copy(out_v, out_hbm.at[pl.ds(wid * BPW, BPW)])


_Q = np.zeros((D, 64), np.float32)
for _m in range(32):
    _Q[_m, _m] = 1.0
    _Q[32 + _m, _m] = 256.0
    _Q[64 + _m, 32 + _m] = 1.0
    _Q[96 + _m, 32 + _m] = 256.0


def kernel(x, table):
    x2 = x.astype(jnp.int32)
    w = (128.5 - 0.5 * (table @ _Q)).astype(jnp.int32)
    tw = w[:, :32] | (w[:, 32:] << 16)
    return _encode(x2, tw)


# speedup vs baseline: 1.6871x; 1.4895x over previous
import functools

import jax
import jax.numpy as jnp
from jax import lax
from jax.experimental import pallas as pl
from jax.experimental.pallas import tpu as pltpu
from jax.experimental.pallas import tpu_sc as plsc

NC, NS, L = 2, 16, 16
NW = NC * NS
B, SEQ, D = 1024, 200, 128
BPW = B // NW
CHA, CHB = 104, 96
ND = D // L

_mesh = plsc.VectorSubcoreMesh(
    core_axis_name="c", subcore_axis_name="s", num_cores=NC, num_subcores=NS
)


@functools.partial(
    pl.kernel,
    out_type=jax.ShapeDtypeStruct((B, D), jnp.float32),
    mesh=_mesh,
    scratch_types=[
        pltpu.VMEM((BPW * SEQ,), jnp.int32),
        pltpu.VMEM((CHA, D), jnp.float32),
        pltpu.VMEM((CHB, D), jnp.float32),
        pltpu.VMEM((BPW, D), jnp.float32),
        pltpu.SemaphoreType.DMA,
        pltpu.SemaphoreType.DMA,
    ],
)
def _encode(x_hbm, table_hbm, out_hbm, idx_v, rowsa, rowsb, out_v, sema, semb):
    wid = lax.axis_index("s") * NC + lax.axis_index("c")

    pltpu.sync_copy(x_hbm.at[pl.ds(wid * BPW * SEQ, BPW * SEQ)], idx_v)

    zero = tuple(jnp.zeros((L,), jnp.float32) for _ in range(ND))

    def ia(s):
        return idx_v.at[pl.ds(pl.multiple_of(s * SEQ, 8), CHA)]

    def ib(s):
        return idx_v.at[pl.ds(pl.multiple_of(s * SEQ + CHA, 8), CHB)]

    def fire_a(s):
        pltpu.async_copy(table_hbm.at[ia(s)], rowsa, sema)

    def fire_b(s):
        pltpu.async_copy(table_hbm.at[ib(s)], rowsb, semb)

    def wait_a(s):
        pltpu.make_async_copy(table_hbm.at[ia(s)], rowsa, sema).wait()

    def wait_b(s):
        pltpu.make_async_copy(table_hbm.at[ib(s)], rowsb, semb).wait()

    def reduce_chunk(buf, n, acc):
        def row_body(r, a):
            return tuple(a[j] + buf[r, pl.ds(j * L, L)] for j in range(ND))

        return lax.fori_loop(0, n, row_body, acc)

    fire_a(0)
    fire_b(0)

    def sample_body(s, carry):
        wait_a(s)
        acc = reduce_chunk(rowsa, CHA, zero)

        @pl.when(s + 1 < BPW)
        def _():
            fire_a(s + 1)

        wait_b(s)
        acc = reduce_chunk(rowsb, CHB, acc)

        @pl.when(s + 1 < BPW)
        def _():
            fire_b(s + 1)

        for j in range(ND):
            out_v[s, pl.ds(j * L, L)] = acc[j]
        return carry

    lax.fori_loop(0, BPW, sample_body, 0)
    pltpu.sync_copy(out_v, out_hbm.at[pl.ds(wid * BPW, BPW)])


def kernel(x, table):
    return _encode(x.astype(jnp.int32).reshape(-1), table)


# speedup vs baseline: 2.2814x; 1.3523x over previous
import functools

import jax
import jax.numpy as jnp
from jax import lax
from jax.experimental import pallas as pl
from jax.experimental.pallas import tpu as pltpu
from jax.experimental.pallas import tpu_sc as plsc

NC, NS, L = 2, 16, 16
NW = NC * NS
B, SEQ, D = 1024, 200, 128
BPW = B // NW
CHA, CHB = 104, 96
ND = D // L

_mesh = plsc.VectorSubcoreMesh(
    core_axis_name="c", subcore_axis_name="s", num_cores=NC, num_subcores=NS
)


@functools.partial(
    pl.kernel,
    out_type=jax.ShapeDtypeStruct((B, D), jnp.float32),
    mesh=_mesh,
    scratch_types=[
        pltpu.VMEM((BPW * SEQ,), jnp.int32),
        pltpu.VMEM((CHA, D), jnp.float32),
        pltpu.VMEM((CHB, D), jnp.float32),
        pltpu.VMEM((CHA, D), jnp.float32),
        pltpu.VMEM((CHB, D), jnp.float32),
        pltpu.VMEM((BPW, D), jnp.float32),
        pltpu.SemaphoreType.DMA,
        pltpu.SemaphoreType.DMA,
        pltpu.SemaphoreType.DMA,
        pltpu.SemaphoreType.DMA,
    ],
)
def _encode(x_hbm, table_hbm, out_hbm, idx_v, ra0, rb0, ra1, rb1, out_v,
            sa0, sb0, sa1, sb1):
    wid = lax.axis_index("s") * NC + lax.axis_index("c")

    pltpu.sync_copy(x_hbm.at[pl.ds(wid * BPW * SEQ, BPW * SEQ)], idx_v)

    zero = tuple(jnp.zeros((L,), jnp.float32) for _ in range(ND))
    ras, rbs = (ra0, ra1), (rb0, rb1)
    sas, sbs = (sa0, sa1), (sb0, sb1)

    def ia(s):
        return idx_v.at[pl.ds(pl.multiple_of(s * SEQ, 8), CHA)]

    def ib(s):
        return idx_v.at[pl.ds(pl.multiple_of(s * SEQ + CHA, 8), CHB)]

    def fire_a(s, h):
        pltpu.async_copy(table_hbm.at[ia(s)], ras[h], sas[h])

    def fire_b(s, h):
        pltpu.async_copy(table_hbm.at[ib(s)], rbs[h], sbs[h])

    def wait_a(s, h):
        pltpu.make_async_copy(table_hbm.at[ia(s)], ras[h], sas[h]).wait()

    def wait_b(s, h):
        pltpu.make_async_copy(table_hbm.at[ib(s)], rbs[h], sbs[h]).wait()

    def reduce_chunk(buf, n, acc):
        def row_body(r, a):
            return tuple(a[j] + buf[r, pl.ds(j * L, L)] for j in range(ND))

        return lax.fori_loop(0, n, row_body, acc)

    fire_a(0, 0)
    fire_b(0, 0)
    fire_a(1, 1)
    fire_b(1, 1)

    def pair_body(i, carry):
        for h in range(2):
            s = 2 * i + h
            wait_a(s, h)
            acc = reduce_chunk(ras[h], CHA, zero)

            @pl.when(s + 2 < BPW)
            def _():
                fire_a(s + 2, h)

            wait_b(s, h)
            acc = reduce_chunk(rbs[h], CHB, acc)

            @pl.when(s + 2 < BPW)
            def _():
                fire_b(s + 2, h)

            for j in range(ND):
                out_v[s, pl.ds(j * L, L)] = acc[j]
        return carry

    lax.fori_loop(0, BPW // 2, pair_body, 0)
    pltpu.sync_copy(out_v, out_hbm.at[pl.ds(wid * BPW, BPW)])


def kernel(x, table):
    return _encode(x.astype(jnp.int32).reshape(-1), table)
